# baseline (speedup 1.0000x reference)
import functools

import jax
import jax.numpy as jnp
from jax.experimental import pallas as pl
from jax.experimental.pallas import tpu as pltpu
from jax.experimental.pallas import tpu_sc as plsc

NE = 64
H = 1024
I = 768
T = 2048
A = 2 * T
BM = 128
NT = 96
S = NT * BM


def _prep_kernel(x_ref, gw_ref, pos_ref, wf_ref, te_ref):
    x = x_ref[...]
    gw = gw_ref[...]
    logits = jax.lax.dot_general(
        x, gw, (((1,), (1,)), ((), ())), preferred_element_type=jnp.float32
    )
    m = jnp.max(logits, axis=-1, keepdims=True)
    ex = jnp.exp(logits - m)
    p = ex / jnp.sum(ex, axis=-1, keepdims=True)

    col = jax.lax.broadcasted_iota(jnp.int32, (T, NE), 1)
    v1 = jnp.max(p, axis=-1, keepdims=True)
    i1 = jnp.min(jnp.where(p == v1, col, NE), axis=-1, keepdims=True)
    m1 = col == i1
    p2 = jnp.where(m1, -1.0, p)
    v2 = jnp.max(p2, axis=-1, keepdims=True)
    i2 = jnp.min(jnp.where(p2 == v2, col, NE), axis=-1, keepdims=True)
    m2 = col == i2
    s = v1 + v2

    O = jnp.concatenate([m1, m2], axis=0).astype(jnp.float32)
    wf = jnp.concatenate([v1 / s, v2 / s], axis=0)

    c = O
    sh = 1
    while sh < A:
        c = c + jnp.concatenate(
            [jnp.zeros((sh, NE), jnp.float32), c[:-sh]], axis=0
        )
        sh *= 2
    excl = c - O
    counts = c[A - 1:A, :]
    ntiles = jnp.ceil(counts / BM)

    ct = ntiles
    sh = 1
    while sh < NE:
        ct = ct + jnp.concatenate(
            [jnp.zeros((1, sh), jnp.float32), ct[:, :-sh]], axis=1
        )
        sh *= 2
    base_rows = (ct - ntiles) * BM

    pos = jnp.sum(O * (excl + base_rows), axis=1, keepdims=True)
    pos_ref[...] = pos
    wf_ref[...] = wf

    tix = jax.lax.broadcasted_iota(jnp.int32, (128, NE), 0).astype(
        jnp.float32
    )
    te = jnp.sum(jnp.where(ct <= tix, 1.0, 0.0), axis=1, keepdims=True)
    te_ref[...] = jnp.minimum(te, NE - 1)


_SC_MESH = plsc.VectorSubcoreMesh(core_axis_name="c", subcore_axis_name="s")


@functools.partial(
    pl.kernel,
    mesh=_SC_MESH,
    out_type=[
        jax.ShapeDtypeStruct((S,), jnp.float32),
        jax.ShapeDtypeStruct((S,), jnp.float32),
        jax.ShapeDtypeStruct((S,), jnp.int32),
    ],
    scratch_types=[
        pltpu.VMEM((A,), jnp.int32),
        pltpu.VMEM((A,), jnp.float32),
        pltpu.VMEM((S,), jnp.float32),
        pltpu.VMEM((S,), jnp.float32),
        pltpu.VMEM((S,), jnp.int32),
    ],
    compiler_params=pltpu.CompilerParams(needs_layout_passes=False),
)
def _sc_dispatch(
    pos_hbm, wf_hbm, stok_hbm, sw_hbm, stoki_hbm, pos_v, wf_v, stok_v, sw_v,
    stoki_v,
):
    wid = jax.lax.axis_index("s") * 2 + jax.lax.axis_index("c")

    @pl.when(wid == 0)
    def _():
        pltpu.sync_copy(pos_hbm, pos_v)
        pltpu.sync_copy(wf_hbm, wf_v)
        z = jnp.zeros((16,), jnp.float32)
        zi = jnp.zeros((16,), jnp.int32)

        def zero_body(j, carry):
            stok_v[pl.ds(j * 16, 16)] = z
            sw_v[pl.ds(j * 16, 16)] = z
            stoki_v[pl.ds(j * 16, 16)] = zi
            return carry

        jax.lax.fori_loop(0, S // 16, zero_body, 0)
        lane = jax.lax.broadcasted_iota(jnp.int32, (16,), 0)

        def scat_body(j, carry):
            a = j * 16 + lane
            idx = pos_v[pl.ds(j * 16, 16)]
            tok = jnp.bitwise_and(a, T - 1)
            plsc.store_scatter(stok_v, [idx], tok.astype(jnp.float32))
            plsc.store_scatter(sw_v, [idx], wf_v[pl.ds(j * 16, 16)])
            plsc.store_scatter(stoki_v, [idx], tok)
            return carry

        jax.lax.fori_loop(0, A // 16, scat_body, 0)
        pltpu.sync_copy(stok_v, stok_hbm)
        pltpu.sync_copy(sw_v, sw_hbm)
        pltpu.sync_copy(stoki_v, stoki_hbm)


NW = 32
RW = S // NW
CH = 64
NCH = RW // CH


@functools.partial(
    pl.kernel,
    mesh=_SC_MESH,
    out_type=jax.ShapeDtypeStruct((S, H), jnp.float32),
    scratch_types=[
        pltpu.VMEM((CH,), jnp.int32),
        pltpu.VMEM((CH, H), jnp.float32),
        pltpu.SemaphoreType.DMA,
    ],
    compiler_params=pltpu.CompilerParams(needs_layout_passes=False),
)
def _sc_gather(stoki_hbm, x_hbm, xs_hbm, idx_v, rows_v, sem):
    wid = jax.lax.axis_index("s") * 2 + jax.lax.axis_index("c")
    base = wid * RW

    def body(ch, carry):
        off = base + ch * CH
        pltpu.sync_copy(stoki_hbm.at[pl.ds(off, CH)], idx_v)
        pltpu.async_copy(x_hbm.at[idx_v], rows_v, sem).wait()
        pltpu.sync_copy(rows_v, xs_hbm.at[pl.ds(off, CH)])
        return carry

    jax.lax.fori_loop(0, NCH, body, 0)


def _main_kernel(te_ref, sw_ref, xs_ref, gu_ref, dp_ref, os_ref):
    xg = xs_ref[...]
    guw = gu_ref[0].astype(jnp.bfloat16)
    gu = jax.lax.dot_general(
        xg.astype(jnp.bfloat16), guw, (((1,), (1,)), ((), ())),
        preferred_element_type=jnp.float32,
    )
    g = gu[:, :I]
    u = gu[:, I:]
    h = (g * jax.lax.logistic(g)) * u
    dpw = dp_ref[0].astype(jnp.bfloat16)
    o = jax.lax.dot_general(
        h.astype(jnp.bfloat16), dpw, (((1,), (1,)), ((), ())),
        preferred_element_type=jnp.float32,
    )
    o = o * sw_ref[0].T
    os_ref[...] = o.astype(jnp.bfloat16)


KC = 1024
NKC = S // KC


def _combine_kernel(stok_ref, os_ref, out_ref):
    k = pl.program_id(0)

    @pl.when(k == 0)
    def _():
        out_ref[...] = jnp.zeros_like(out_ref)

    st = stok_ref[0]
    toki = jax.lax.broadcasted_iota(jnp.int32, (T, 1), 0).astype(jnp.float32)
    Sc = jnp.where(st == toki, 1.0, 0.0).astype(jnp.bfloat16)
    out_ref[...] += jax.lax.dot_general(
        Sc, os_ref[...], (((1,), (0,)), ((), ())),
        preferred_element_type=jnp.float32,
    )


def kernel(hidden_states, gate_w, gate_up_proj, down_proj):
    pos, wf, te = pl.pallas_call(
        _prep_kernel,
        out_shape=[
            jax.ShapeDtypeStruct((A, 1), jnp.float32),
            jax.ShapeDtypeStruct((A, 1), jnp.float32),
            jax.ShapeDtypeStruct((128, 1), jnp.float32),
        ],
        in_specs=[
            pl.BlockSpec((T, H), lambda: (0, 0)),
            pl.BlockSpec((NE, H), lambda: (0, 0)),
        ],
        out_specs=[
            pl.BlockSpec((A, 1), lambda: (0, 0)),
            pl.BlockSpec((A, 1), lambda: (0, 0)),
            pl.BlockSpec((128, 1), lambda: (0, 0)),
        ],
    )(hidden_states, gate_w)

    stok_f, sw_f, stoki = _sc_dispatch(
        pos.reshape(A).astype(jnp.int32), wf.reshape(A)
    )
    stok = stok_f.reshape(NT, 1, BM)
    sw = sw_f.reshape(NT, 1, BM)

    xs = _sc_gather(stoki, hidden_states)

    te_i32 = te.reshape(128).astype(jnp.int32)

    grid_spec = pltpu.PrefetchScalarGridSpec(
        num_scalar_prefetch=1,
        grid=(NT,),
        in_specs=[
            pl.BlockSpec((1, 1, BM), lambda t, te: (t, 0, 0)),
            pl.BlockSpec((BM, H), lambda t, te: (t, 0)),
            pl.BlockSpec((1, 2 * I, H), lambda t, te: (te[t], 0, 0)),
            pl.BlockSpec((1, H, I), lambda t, te: (te[t], 0, 0)),
        ],
        out_specs=pl.BlockSpec((BM, H), lambda t, te: (t, 0)),
    )

    os = pl.pallas_call(
        _main_kernel,
        grid_spec=grid_spec,
        out_shape=jax.ShapeDtypeStruct((S, H), jnp.bfloat16),
        compiler_params=pltpu.CompilerParams(
            dimension_semantics=("arbitrary",),
        ),
    )(te_i32, sw, xs, gate_up_proj, down_proj)

    out = pl.pallas_call(
        _combine_kernel,
        grid=(NKC,),
        out_shape=jax.ShapeDtypeStruct((T, H), jnp.float32),
        in_specs=[
            pl.BlockSpec((1, 1, KC), lambda k: (k, 0, 0)),
            pl.BlockSpec((KC, H), lambda k: (k, 0)),
        ],
        out_specs=pl.BlockSpec((T, H), lambda k: (0, 0)),
        compiler_params=pltpu.CompilerParams(
            dimension_semantics=("arbitrary",),
        ),
    )(stok.reshape(NKC, 1, KC), os)
    return out


# speedup vs baseline: 2.4405x; 2.4405x over previous
import functools

import jax
import jax.numpy as jnp
from jax.experimental import pallas as pl
from jax.experimental.pallas import tpu as pltpu
from jax.experimental.pallas import tpu_sc as plsc

NE = 64
H = 1024
I = 768
T = 2048
A = 2 * T
BM = 128
NT = 96
S = NT * BM


def _prep_kernel(x_ref, gw_ref, pos_ref, wf_ref, te_ref, nu_ref):
    x = x_ref[...]
    gw = gw_ref[...]
    logits = jax.lax.dot_general(
        x, gw, (((1,), (1,)), ((), ())), preferred_element_type=jnp.float32
    )
    m = jnp.max(logits, axis=-1, keepdims=True)
    ex = jnp.exp(logits - m)
    p = ex / jnp.sum(ex, axis=-1, keepdims=True)

    col = jax.lax.broadcasted_iota(jnp.int32, (T, NE), 1)
    v1 = jnp.max(p, axis=-1, keepdims=True)
    i1 = jnp.min(jnp.where(p == v1, col, NE), axis=-1, keepdims=True)
    m1 = col == i1
    p2 = jnp.where(m1, -1.0, p)
    v2 = jnp.max(p2, axis=-1, keepdims=True)
    i2 = jnp.min(jnp.where(p2 == v2, col, NE), axis=-1, keepdims=True)
    m2 = col == i2
    s = v1 + v2

    O = jnp.concatenate([m1, m2], axis=0).astype(jnp.float32)
    wf = jnp.concatenate([v1 / s, v2 / s], axis=0)

    c = O
    sh = 1
    while sh < A:
        c = c + jnp.concatenate(
            [jnp.zeros((sh, NE), jnp.float32), c[:-sh]], axis=0
        )
        sh *= 2
    excl = c - O
    counts = c[A - 1:A, :]
    ntiles = jnp.ceil(counts / BM)

    ct = ntiles
    sh = 1
    while sh < NE:
        ct = ct + jnp.concatenate(
            [jnp.zeros((1, sh), jnp.float32), ct[:, :-sh]], axis=1
        )
        sh *= 2
    base_rows = (ct - ntiles) * BM

    pos = jnp.sum(O * (excl + base_rows), axis=1, keepdims=True)
    pos_ref[...] = pos
    wf_ref[...] = wf

    tix = jax.lax.broadcasted_iota(jnp.int32, (128, NE), 0).astype(
        jnp.float32
    )
    te = jnp.sum(jnp.where(ct <= tix, 1.0, 0.0), axis=1, keepdims=True)
    te_ref[...] = jnp.minimum(te, NE - 1)
    nu_ref[...] = jnp.broadcast_to(ct[0:1, NE - 1:NE], (8, 1))


_SC_MESH = plsc.VectorSubcoreMesh(core_axis_name="c", subcore_axis_name="s")


@functools.partial(
    pl.kernel,
    mesh=_SC_MESH,
    out_type=[
        jax.ShapeDtypeStruct((S,), jnp.float32),
        jax.ShapeDtypeStruct((S,), jnp.float32),
    ],
    scratch_types=[
        pltpu.VMEM((A,), jnp.int32),
        pltpu.VMEM((A,), jnp.float32),
        pltpu.VMEM((S,), jnp.float32),
        pltpu.VMEM((S,), jnp.float32),
    ],
    compiler_params=pltpu.CompilerParams(needs_layout_passes=False),
)
def _sc_dispatch(pos_hbm, wf_hbm, stok_hbm, sw_hbm, pos_v, wf_v, stok_v, sw_v):
    wid = jax.lax.axis_index("s") * 2 + jax.lax.axis_index("c")

    @pl.when(wid == 0)
    def _():
        pltpu.sync_copy(pos_hbm, pos_v)
        pltpu.sync_copy(wf_hbm, wf_v)
        z = jnp.zeros((16,), jnp.float32)

        def zero_body(j, carry):
            stok_v[pl.ds(j * 16, 16)] = z
            sw_v[pl.ds(j * 16, 16)] = z
            return carry

        jax.lax.fori_loop(0, S // 16, zero_body, 0)
        lane = jax.lax.broadcasted_iota(jnp.int32, (16,), 0)

        def scat_body(j, carry):
            a = j * 16 + lane
            idx = pos_v[pl.ds(j * 16, 16)]
            tok = jnp.bitwise_and(a, T - 1)
            plsc.store_scatter(stok_v, [idx], tok.astype(jnp.float32))
            plsc.store_scatter(sw_v, [idx], wf_v[pl.ds(j * 16, 16)])
            return carry

        jax.lax.fori_loop(0, A // 16, scat_body, 0)
        pltpu.sync_copy(stok_v, stok_hbm)
        pltpu.sync_copy(sw_v, sw_hbm)


def _main_kernel(te_ref, nu_ref, stok_ref, sw_ref, xb_ref, gu_ref, dp_ref,
                 os_ref):
    t = pl.program_id(0)

    @pl.when(t < nu_ref[0])
    def _():
        stc = stok_ref[0].T
        cols = jax.lax.broadcasted_iota(jnp.int32, (BM, T), 1).astype(
            jnp.float32
        )
        G = jnp.where(stc == cols, 1.0, 0.0).astype(jnp.bfloat16)

        xg = jax.lax.dot_general(
            G, xb_ref[...], (((1,), (0,)), ((), ())),
            preferred_element_type=jnp.float32,
        )
        guw = gu_ref[0].astype(jnp.bfloat16)
        gu = jax.lax.dot_general(
            xg.astype(jnp.bfloat16), guw, (((1,), (1,)), ((), ())),
            preferred_element_type=jnp.float32,
        )
        g = gu[:, :I]
        u = gu[:, I:]
        h = (g * jax.lax.logistic(g)) * u
        dpw = dp_ref[0].astype(jnp.bfloat16)
        o = jax.lax.dot_general(
            h.astype(jnp.bfloat16), dpw, (((1,), (1,)), ((), ())),
            preferred_element_type=jnp.float32,
        )
        o = o * sw_ref[0].T
        os_ref[...] = o.astype(jnp.bfloat16)

    @pl.when(t >= nu_ref[0])
    def _():
        os_ref[...] = jnp.zeros_like(os_ref)


KC = 1024
NKC = S // KC


def _combine_kernel(nu_ref, stok_ref, os_ref, out_ref):
    k = pl.program_id(0)

    @pl.when(k == 0)
    def _():
        out_ref[...] = jnp.zeros_like(out_ref)

    @pl.when(k * KC < nu_ref[0] * BM)
    def _():
        st = stok_ref[0]
        toki = jax.lax.broadcasted_iota(jnp.int32, (T, 1), 0).astype(
            jnp.float32
        )
        Sc = jnp.where(st == toki, 1.0, 0.0).astype(jnp.bfloat16)
        out_ref[...] += jax.lax.dot_general(
            Sc, os_ref[...], (((1,), (0,)), ((), ())),
            preferred_element_type=jnp.float32,
        )


def kernel(hidden_states, gate_w, gate_up_proj, down_proj):
    pos, wf, te, nu = pl.pallas_call(
        _prep_kernel,
        out_shape=[
            jax.ShapeDtypeStruct((A, 1), jnp.float32),
            jax.ShapeDtypeStruct((A, 1), jnp.float32),
            jax.ShapeDtypeStruct((128, 1), jnp.float32),
            jax.ShapeDtypeStruct((8, 1), jnp.float32),
        ],
        in_specs=[
            pl.BlockSpec((T, H), lambda: (0, 0)),
            pl.BlockSpec((NE, H), lambda: (0, 0)),
        ],
        out_specs=[
            pl.BlockSpec((A, 1), lambda: (0, 0)),
            pl.BlockSpec((A, 1), lambda: (0, 0)),
            pl.BlockSpec((128, 1), lambda: (0, 0)),
            pl.BlockSpec((8, 1), lambda: (0, 0)),
        ],
    )(hidden_states, gate_w)

    stok_f, sw_f = _sc_dispatch(
        pos.reshape(A).astype(jnp.int32), wf.reshape(A)
    )
    stok = stok_f.reshape(NT, 1, BM)
    sw = sw_f.reshape(NT, 1, BM)

    te_i32 = te.reshape(128).astype(jnp.int32)
    nu_i32 = nu.reshape(8)[:1].astype(jnp.int32)
    xb = hidden_states.astype(jnp.bfloat16)

    grid_spec = pltpu.PrefetchScalarGridSpec(
        num_scalar_prefetch=2,
        grid=(NT,),
        in_specs=[
            pl.BlockSpec((1, 1, BM), lambda t, te, nu: (t, 0, 0)),
            pl.BlockSpec((1, 1, BM), lambda t, te, nu: (t, 0, 0)),
            pl.BlockSpec((T, H), lambda t, te, nu: (0, 0)),
            pl.BlockSpec((1, 2 * I, H), lambda t, te, nu: (te[t], 0, 0)),
            pl.BlockSpec((1, H, I), lambda t, te, nu: (te[t], 0, 0)),
        ],
        out_specs=pl.BlockSpec((BM, H), lambda t, te, nu: (t, 0)),
    )

    os = pl.pallas_call(
        _main_kernel,
        grid_spec=grid_spec,
        out_shape=jax.ShapeDtypeStruct((S, H), jnp.bfloat16),
        compiler_params=pltpu.CompilerParams(
            dimension_semantics=("arbitrary",),
        ),
    )(te_i32, nu_i32, stok, sw, xb, gate_up_proj, down_proj)

    combine_spec = pltpu.PrefetchScalarGridSpec(
        num_scalar_prefetch=1,
        grid=(NKC,),
        in_specs=[
            pl.BlockSpec((1, 1, KC), lambda k, nu: (k, 0, 0)),
            pl.BlockSpec((KC, H), lambda k, nu: (k, 0)),
        ],
        out_specs=pl.BlockSpec((T, H), lambda k, nu: (0, 0)),
    )

    out = pl.pallas_call(
        _combine_kernel,
        grid_spec=combine_spec,
        out_shape=jax.ShapeDtypeStruct((T, H), jnp.float32),
        compiler_params=pltpu.CompilerParams(
            dimension_semantics=("arbitrary",),
        ),
    )(nu_i32, stok.reshape(NKC, 1, KC), os)
    return out


# speedup vs baseline: 2.5789x; 1.0567x over previous
import functools

import jax
import jax.numpy as jnp
from jax.experimental import pallas as pl
from jax.experimental.pallas import tpu as pltpu
from jax.experimental.pallas import tpu_sc as plsc

NE = 64
H = 1024
I = 768
T = 2048
A = 2 * T
BM = 128
NT = 96
S = NT * BM


def _prep_kernel(x_ref, gw_ref, pos_ref, wf_ref, te_ref, nu_ref):
    x = x_ref[...]
    gw = gw_ref[...]
    logits = jax.lax.dot_general(
        x, gw, (((1,), (1,)), ((), ())), preferred_element_type=jnp.float32
    )
    m = jnp.max(logits, axis=-1, keepdims=True)
    ex = jnp.exp(logits - m)
    p = ex / jnp.sum(ex, axis=-1, keepdims=True)

    col = jax.lax.broadcasted_iota(jnp.int32, (T, NE), 1)
    v1 = jnp.max(p, axis=-1, keepdims=True)
    i1 = jnp.min(jnp.where(p == v1, col, NE), axis=-1, keepdims=True)
    m1 = col == i1
    p2 = jnp.where(m1, -1.0, p)
    v2 = jnp.max(p2, axis=-1, keepdims=True)
    i2 = jnp.min(jnp.where(p2 == v2, col, NE), axis=-1, keepdims=True)
    m2 = col == i2
    s = v1 + v2

    O = jnp.concatenate([m1, m2], axis=0).astype(jnp.float32)
    wf = jnp.concatenate([v1 / s, v2 / s], axis=0)

    c = O
    sh = 1
    while sh < A:
        c = c + jnp.concatenate(
            [jnp.zeros((sh, NE), jnp.float32), c[:-sh]], axis=0
        )
        sh *= 2
    excl = c - O
    counts = c[A - 1:A, :]
    ntiles = jnp.ceil(counts / BM)

    ct = ntiles
    sh = 1
    while sh < NE:
        ct = ct + jnp.concatenate(
            [jnp.zeros((1, sh), jnp.float32), ct[:, :-sh]], axis=1
        )
        sh *= 2
    base_rows = (ct - ntiles) * BM

    pos = jnp.sum(O * (excl + base_rows), axis=1, keepdims=True)
    pos_ref[...] = pos
    wf_ref[...] = wf

    tix = jax.lax.broadcasted_iota(jnp.int32, (128, NE), 0).astype(
        jnp.float32
    )
    te = jnp.sum(jnp.where(ct <= tix, 1.0, 0.0), axis=1, keepdims=True)
    te_ref[...] = jnp.minimum(te, NE - 1)
    nu_ref[...] = jnp.broadcast_to(ct[0:1, NE - 1:NE], (8, 1))


_SC_MESH = plsc.VectorSubcoreMesh(core_axis_name="c", subcore_axis_name="s")


@functools.partial(
    pl.kernel,
    mesh=_SC_MESH,
    out_type=[
        jax.ShapeDtypeStruct((S,), jnp.float32),
        jax.ShapeDtypeStruct((S,), jnp.float32),
    ],
    scratch_types=[
        pltpu.VMEM((A,), jnp.int32),
        pltpu.VMEM((A,), jnp.float32),
        pltpu.VMEM((S,), jnp.float32),
        pltpu.VMEM((S,), jnp.float32),
    ],
    compiler_params=pltpu.CompilerParams(needs_layout_passes=False),
)
def _sc_dispatch(pos_hbm, wf_hbm, stok_hbm, sw_hbm, pos_v, wf_v, stok_v, sw_v):
    wid = jax.lax.axis_index("s") * 2 + jax.lax.axis_index("c")

    @pl.when(wid == 0)
    def _():
        pltpu.sync_copy(pos_hbm, pos_v)
        pltpu.sync_copy(wf_hbm, wf_v)
        z = jnp.zeros((16,), jnp.float32)

        def zero_body(j, carry):
            stok_v[pl.ds(j * 16, 16)] = z
            sw_v[pl.ds(j * 16, 16)] = z
            return carry

        jax.lax.fori_loop(0, S // 16, zero_body, 0)
        lane = jax.lax.broadcasted_iota(jnp.int32, (16,), 0)

        def scat_body(j, carry):
            a = j * 16 + lane
            idx = pos_v[pl.ds(j * 16, 16)]
            tok = jnp.bitwise_and(a, T - 1)
            plsc.store_scatter(stok_v, [idx], tok.astype(jnp.float32))
            plsc.store_scatter(sw_v, [idx], wf_v[pl.ds(j * 16, 16)])
            return carry

        jax.lax.fori_loop(0, A // 16, scat_body, 0)
        pltpu.sync_copy(stok_v, stok_hbm)
        pltpu.sync_copy(sw_v, sw_hbm)


def _main_kernel(te_ref, nu_ref, stok_ref, sw_ref, xb_ref, gu_ref, dp_ref,
                 out_ref):
    t = pl.program_id(0)

    @pl.when(t == 0)
    def _():
        out_ref[...] = jnp.zeros_like(out_ref)

    @pl.when(t < nu_ref[0])
    def _():
        stc = stok_ref[0].T
        cols = jax.lax.broadcasted_iota(jnp.int32, (BM, T), 1).astype(
            jnp.float32
        )
        G = jnp.where(stc == cols, 1.0, 0.0).astype(jnp.bfloat16)

        xg = jax.lax.dot_general(
            G, xb_ref[...], (((1,), (0,)), ((), ())),
            preferred_element_type=jnp.float32,
        )
        guw = gu_ref[0].astype(jnp.bfloat16)
        gu = jax.lax.dot_general(
            xg.astype(jnp.bfloat16), guw, (((1,), (1,)), ((), ())),
            preferred_element_type=jnp.float32,
        )
        g = gu[:, :I]
        u = gu[:, I:]
        h = (g * jax.lax.logistic(g)) * u
        dpw = dp_ref[0].astype(jnp.bfloat16)
        o = jax.lax.dot_general(
            h.astype(jnp.bfloat16), dpw, (((1,), (1,)), ((), ())),
            preferred_element_type=jnp.float32,
        )
        o = o * sw_ref[0].T
        toki = jax.lax.broadcasted_iota(jnp.int32, (T, BM), 0).astype(
            jnp.float32
        )
        Gt = jnp.where(stok_ref[0] == toki, 1.0, 0.0).astype(
            jnp.bfloat16
        )
        out_ref[...] += jax.lax.dot_general(
            Gt, o.astype(jnp.bfloat16), (((1,), (0,)), ((), ())),
            preferred_element_type=jnp.float32,
        )


def kernel(hidden_states, gate_w, gate_up_proj, down_proj):
    pos, wf, te, nu = pl.pallas_call(
        _prep_kernel,
        out_shape=[
            jax.ShapeDtypeStruct((A, 1), jnp.float32),
            jax.ShapeDtypeStruct((A, 1), jnp.float32),
            jax.ShapeDtypeStruct((128, 1), jnp.float32),
            jax.ShapeDtypeStruct((8, 1), jnp.float32),
        ],
        in_specs=[
            pl.BlockSpec((T, H), lambda: (0, 0)),
            pl.BlockSpec((NE, H), lambda: (0, 0)),
        ],
        out_specs=[
            pl.BlockSpec((A, 1), lambda: (0, 0)),
            pl.BlockSpec((A, 1), lambda: (0, 0)),
            pl.BlockSpec((128, 1), lambda: (0, 0)),
            pl.BlockSpec((8, 1), lambda: (0, 0)),
        ],
    )(hidden_states, gate_w)

    stok_f, sw_f = _sc_dispatch(
        pos.reshape(A).astype(jnp.int32), wf.reshape(A)
    )
    stok = stok_f.reshape(NT, 1, BM)
    sw = sw_f.reshape(NT, 1, BM)

    te_i32 = te.reshape(128).astype(jnp.int32)
    nu_i32 = nu.reshape(8)[:1].astype(jnp.int32)
    xb = hidden_states.astype(jnp.bfloat16)

    grid_spec = pltpu.PrefetchScalarGridSpec(
        num_scalar_prefetch=2,
        grid=(NT,),
        in_specs=[
            pl.BlockSpec((1, 1, BM), lambda t, te, nu: (t, 0, 0)),
            pl.BlockSpec((1, 1, BM), lambda t, te, nu: (t, 0, 0)),
            pl.BlockSpec((T, H), lambda t, te, nu: (0, 0)),
            pl.BlockSpec((1, 2 * I, H), lambda t, te, nu: (te[t], 0, 0)),
            pl.BlockSpec((1, H, I), lambda t, te, nu: (te[t], 0, 0)),
        ],
        out_specs=pl.BlockSpec((T, H), lambda t, te, nu: (0, 0)),
    )

    out = pl.pallas_call(
        _main_kernel,
        grid_spec=grid_spec,
        out_shape=jax.ShapeDtypeStruct((T, H), jnp.float32),
        compiler_params=pltpu.CompilerParams(
            dimension_semantics=("arbitrary",),
        ),
    )(te_i32, nu_i32, stok, sw, xb, gate_up_proj, down_proj)
    return out
